# baseline (speedup 1.0000x reference)
import functools

import jax
import jax.numpy as jnp
from jax import lax
from jax.experimental import pallas as pl
from jax.experimental.pallas import tpu as pltpu
from jax.experimental.pallas import tpu_sc as plsc

N = 10000
E = 320000
D = 128
DH = D // 2
CW = 16

NC = 2
NS = 16
NW = NC * NS
EPW = E // NW
EPS = E // NS
CH = 50
NBUF = 4
SB = 40
NSB = EPS // (SB * CH)
NP = 10240
RPT = NP // NS


def _sc_agg_body(with_cnt, xh_hbm, src_hbm, dst_hbm, zacc_hbm, dstc_hbm,
                 zc_hbm, on_hbm,
                 acc_out, cnt_out,
                 sbs0, sbs1, sbd0, sbd1, rows0, rows1, rows2, rows3,
                 x_sh, acc_sh,
                 gs0, gs1, gs2, gs3, ss0, ss1, ss2, ss3,
                 is0, is1, is2, is3,
                 dstcbuf, onesbuf, cnt_sh,
                 cs0, cs1, cs2, cs3, cs4, cs5, cs6, cs7):
    c = lax.axis_index("c")
    s = lax.axis_index("s")
    rows = (rows0, rows1, rows2, rows3)
    gsem = (gs0, gs1, gs2, gs3)
    ssem = (ss0, ss1, ss2, ss3)
    csem = (cs0, cs1, cs2, cs3, cs4, cs5, cs6, cs7)
    sbsrc = (sbs0, sbs1)
    sbdst = (sbd0, sbd1)
    isem = ((is0, is1), (is2, is3))

    pltpu.sync_copy(xh_hbm.at[c, pl.ds(s * RPT, RPT)],
                    x_sh.at[pl.ds(s * RPT, RPT)])
    pltpu.sync_copy(zacc_hbm, acc_sh.at[pl.ds(s * RPT, RPT)])
    if with_cnt:
        wid = s * NC + c
        pltpu.sync_copy(zc_hbm, cnt_sh.at[pl.ds(s * RPT, RPT)])
        pltpu.sync_copy(on_hbm, onesbuf)
        pltpu.sync_copy(dstc_hbm.at[wid], dstcbuf)
    plsc.subcore_barrier()

    def stage(sb, p):
        return (pltpu.async_copy(src_hbm.at[s, sb], sbsrc[p], isem[p][0]),
                pltpu.async_copy(dst_hbm.at[s, sb], sbdst[p], isem[p][1]))

    def make_chunk(p):
        def chunk(k, carry):
            k0 = NBUF * k
            gcp = [pltpu.async_copy(x_sh.at[sbsrc[p].at[k0 + b]], rows[b],
                                    gsem[b])
                   for b in range(NBUF)]
            scp = []
            for b in range(NBUF):
                gcp[b].wait()
                scp.append(pltpu.async_copy(rows[b],
                                            acc_sh.at[sbdst[p].at[k0 + b]],
                                            ssem[b], add=True))
            for b in range(NBUF):
                scp[b].wait()
            return carry
        return chunk

    CPS = NCHUNKC // NSB
    cnt_pending = []
    pending = stage(0, 0)
    for sb in range(NSB):
        p = sb % 2
        for cp in pending:
            cp.wait()
        if sb + 1 < NSB:
            nxt = stage(sb + 1, 1 - p)
        if with_cnt:
            for cp in cnt_pending:
                cp.wait()
            cnt_pending = [
                pltpu.async_copy(onesbuf,
                                 cnt_sh.at[dstcbuf.at[sb * CPS + j]],
                                 csem[j], add=True)
                for j in range(CPS)]
        lax.fori_loop(0, SB // NBUF, make_chunk(p), 0)
        if sb + 1 < NSB:
            pending = nxt
    for cp in cnt_pending:
        cp.wait()
    plsc.subcore_barrier()

    pltpu.sync_copy(acc_sh.at[pl.ds(s * RPT, RPT)],
                    acc_out.at[c, pl.ds(s * RPT, RPT)])
    if with_cnt:
        pltpu.sync_copy(cnt_sh.at[pl.ds(s * RPT, RPT)],
                        cnt_out.at[c, pl.ds(s * RPT, RPT)])


def _make_sc_agg(with_cnt):
    return pl.kernel(
        functools.partial(_sc_agg_body, with_cnt),
        out_type=(jax.ShapeDtypeStruct((NC, NP, DH), jnp.float32),
                  jax.ShapeDtypeStruct((NC, NP, CW), jnp.float32)),
        mesh=plsc.VectorSubcoreMesh(core_axis_name="c",
                                    subcore_axis_name="s"),
        scratch_types=(
            [pltpu.VMEM((SB, CH), jnp.int32)] * 4
            + [pltpu.VMEM((CH, DH), jnp.float32)] * NBUF
            + [pltpu.VMEM_SHARED((NP, DH), jnp.float32)]
            + [pltpu.VMEM_SHARED((NP, DH), jnp.float32)]
            + [pltpu.SemaphoreType.DMA] * (2 * NBUF + 4)
            + [pltpu.VMEM((NCHUNKC, CHC), jnp.int32)]
            + [pltpu.VMEM((CHC, CW), jnp.float32)]
            + [pltpu.VMEM_SHARED((NP, CW), jnp.float32)]
            + [pltpu.SemaphoreType.DMA] * 8
        ),
        compiler_params=pltpu.CompilerParams(use_tc_tiling_on_sc=False),
    )


CHC = 125
NCHUNKC = EPW // CHC

_sc_agg_cnt = _make_sc_agg(True)
_sc_agg = _make_sc_agg(False)


def _tc_layer_body(split_out, acc_ref, cnt_ref, x_ref, wl_ref, wr_ref, b_ref,
                   o_ref):
    acc = jnp.concatenate([acc_ref[0], acc_ref[1]], axis=-1)
    xin = jnp.concatenate([x_ref[0], x_ref[1]], axis=-1)
    cnt = cnt_ref[0, :, 0] + cnt_ref[1, :, 0]
    inv = 1.0 / jnp.maximum(cnt, 1.0)
    mean = acc * inv[:, None]
    hl = lax.dot_general(mean, wl_ref[...], (((1,), (1,)), ((), ())),
                         preferred_element_type=jnp.float32)
    hr = lax.dot_general(xin, wr_ref[...], (((1,), (1,)), ((), ())),
                         preferred_element_type=jnp.float32)
    act = jnp.maximum(hl + hr + b_ref[...], 0.0)
    if split_out:
        o_ref[0] = act[:, :DH]
        o_ref[1] = act[:, DH:]
    else:
        o_ref[...] = act


def _tc_layer(acc, cnt, xh, wl, wr, b, split_out):
    if split_out:
        rows, tcb = NP, 2048
        out_shape = jax.ShapeDtypeStruct((NC, NP, DH), jnp.float32)
        out_spec = pl.BlockSpec((NC, tcb, DH), lambda i: (0, i, 0))
    else:
        rows, tcb = N, 2000
        out_shape = jax.ShapeDtypeStruct((N, D), jnp.float32)
        out_spec = pl.BlockSpec((tcb, D), lambda i: (i, 0))
    return pl.pallas_call(
        functools.partial(_tc_layer_body, split_out),
        grid=(rows // tcb,),
        in_specs=[
            pl.BlockSpec((NC, tcb, DH), lambda i: (0, i, 0)),
            pl.BlockSpec((NC, tcb, CW), lambda i: (0, i, 0)),
            pl.BlockSpec((NC, tcb, DH), lambda i: (0, i, 0)),
            pl.BlockSpec((D, D), lambda i: (0, 0)),
            pl.BlockSpec((D, D), lambda i: (0, 0)),
            pl.BlockSpec((1, D), lambda i: (0, 0)),
        ],
        out_specs=out_spec,
        out_shape=out_shape,
    )(acc, cnt, xh, wl, wr, b)


def kernel(x, edge_index, W1l, W1r, b1, W2l, W2r, b2):
    src2 = edge_index[0].reshape(NS, NSB, SB, CH)
    dst2 = edge_index[1].reshape(NS, NSB, SB, CH)
    dst3c = edge_index[1].reshape(NW, NCHUNKC, CHC)
    zacc = jnp.zeros((RPT, DH), jnp.float32)
    zc = jnp.zeros((RPT, CW), jnp.float32)
    on1 = jnp.ones((CHC, CW), jnp.float32)

    xp = jnp.zeros((NP, D), jnp.float32).at[:N].set(x)
    xh = xp.reshape(NP, NC, DH).transpose(1, 0, 2)

    acc1, cnt1 = _sc_agg_cnt(xh, src2, dst2, zacc, dst3c, zc, on1)
    hh = _tc_layer(acc1, cnt1, xh, W1l, W1r, b1.reshape(1, D), True)
    acc2, _ = _sc_agg(hh, src2, dst2, zacc, dst3c, zc, on1)
    return _tc_layer(acc2, cnt1, hh, W2l, W2r, b2.reshape(1, D), False)


# speedup vs baseline: 1.0578x; 1.0578x over previous
import functools

import jax
import jax.numpy as jnp
from jax import lax
from jax.experimental import pallas as pl
from jax.experimental.pallas import tpu as pltpu
from jax.experimental.pallas import tpu_sc as plsc

N = 10000
E = 320000
D = 128
CW = 16

NC = 2
NS = 16
NW = NC * NS
EPW = E // NW
CH = 50
NCHUNK = EPW // CH
NBUF = 4
NP = 10240
RPT = NP // NS


def _sc_agg_body(x_hbm, src_hbm, dst_hbm, zacc_hbm,
                 acc_out,
                 srcbuf, dstbuf, rows0, rows1, rows2, rows3,
                 acc_sh,
                 gs0, gs1, gs2, gs3, ss0, ss1, ss2, ss3):
    c = lax.axis_index("c")
    s = lax.axis_index("s")
    wid = s * NC + c
    rows = (rows0, rows1, rows2, rows3)
    gsem = (gs0, gs1, gs2, gs3)
    ssem = (ss0, ss1, ss2, ss3)

    pltpu.sync_copy(zacc_hbm, acc_sh.at[pl.ds(s * RPT, RPT)])
    pltpu.sync_copy(src_hbm.at[wid], srcbuf)
    pltpu.sync_copy(dst_hbm.at[wid], dstbuf)
    plsc.subcore_barrier()

    def chunk(k, carry):
        k0 = NBUF * k
        gcp = [pltpu.async_copy(x_hbm.at[srcbuf.at[k0 + b]], rows[b], gsem[b])
               for b in range(NBUF)]
        scp = []
        for b in range(NBUF):
            gcp[b].wait()
            scp.append(pltpu.async_copy(rows[b], acc_sh.at[dstbuf.at[k0 + b]],
                                        ssem[b], add=True))
        for b in range(NBUF):
            scp[b].wait()
        return carry

    lax.fori_loop(0, NCHUNK // NBUF, chunk, 0)
    plsc.subcore_barrier()

    pltpu.sync_copy(acc_sh.at[pl.ds(s * RPT, RPT)],
                    acc_out.at[c, pl.ds(s * RPT, RPT)])


_sc_agg = pl.kernel(
    _sc_agg_body,
    out_type=jax.ShapeDtypeStruct((NC, NP, D), jnp.float32),
    mesh=plsc.VectorSubcoreMesh(core_axis_name="c", subcore_axis_name="s"),
    scratch_types=(
        [
            pltpu.VMEM((NCHUNK, CH), jnp.int32),
            pltpu.VMEM((NCHUNK, CH), jnp.int32),
        ]
        + [pltpu.VMEM((CH, D), jnp.float32)] * NBUF
        + [pltpu.VMEM_SHARED((NP, D), jnp.float32)]
        + [pltpu.SemaphoreType.DMA] * (2 * NBUF)
    ),
    compiler_params=pltpu.CompilerParams(use_tc_tiling_on_sc=False),
)


CHC = 125
NCHUNKC = EPW // CHC


def _sc_cnt_body(dst_hbm, zc_hbm, on_hbm,
                 cnt_out,
                 dstbuf, onesbuf, cnt_sh, ss0, ss1, ss2, ss3):
    c = lax.axis_index("c")
    s = lax.axis_index("s")
    wid = s * NC + c
    ssem = (ss0, ss1, ss2, ss3)

    pltpu.sync_copy(zc_hbm, cnt_sh.at[pl.ds(s * RPT, RPT)])
    pltpu.sync_copy(on_hbm, onesbuf)
    pltpu.sync_copy(dst_hbm.at[wid], dstbuf)
    plsc.subcore_barrier()

    def chunk(k, carry):
        k0 = NBUF * k
        scp = [pltpu.async_copy(onesbuf, cnt_sh.at[dstbuf.at[k0 + b]],
                                ssem[b], add=True)
               for b in range(NBUF)]
        for cp in scp:
            cp.wait()
        return carry

    lax.fori_loop(0, NCHUNKC // NBUF, chunk, 0)
    plsc.subcore_barrier()

    pltpu.sync_copy(cnt_sh.at[pl.ds(s * RPT, RPT)],
                    cnt_out.at[c, pl.ds(s * RPT, RPT)])


_sc_cnt = pl.kernel(
    _sc_cnt_body,
    out_type=jax.ShapeDtypeStruct((NC, NP, CW), jnp.float32),
    mesh=plsc.VectorSubcoreMesh(core_axis_name="c", subcore_axis_name="s"),
    scratch_types=[
        pltpu.VMEM((NCHUNKC, CHC), jnp.int32),
        pltpu.VMEM((CHC, CW), jnp.float32),
        pltpu.VMEM_SHARED((NP, CW), jnp.float32),
    ] + [pltpu.SemaphoreType.DMA] * NBUF,
    compiler_params=pltpu.CompilerParams(use_tc_tiling_on_sc=False),
)


def _tc_layer_body(acc_ref, cnt_ref, x_ref, wl_ref, wr_ref, b_ref, o_ref):
    acc = acc_ref[0] + acc_ref[1]
    cnt = cnt_ref[0, :, 0] + cnt_ref[1, :, 0]
    inv = 1.0 / jnp.maximum(cnt, 1.0)
    mean = acc * inv[:, None]
    hl = lax.dot_general(mean, wl_ref[...], (((1,), (1,)), ((), ())),
                         preferred_element_type=jnp.float32)
    hr = lax.dot_general(x_ref[...], wr_ref[...], (((1,), (1,)), ((), ())),
                         preferred_element_type=jnp.float32)
    o_ref[...] = jnp.maximum(hl + hr + b_ref[...], 0.0)


_TCB = 2000


def _tc_layer(acc, cnt, x, wl, wr, b):
    grid = (N // _TCB,)
    return pl.pallas_call(
        _tc_layer_body,
        grid=grid,
        in_specs=[
            pl.BlockSpec((NC, _TCB, D), lambda i: (0, i, 0)),
            pl.BlockSpec((NC, _TCB, CW), lambda i: (0, i, 0)),
            pl.BlockSpec((_TCB, D), lambda i: (i, 0)),
            pl.BlockSpec((D, D), lambda i: (0, 0)),
            pl.BlockSpec((D, D), lambda i: (0, 0)),
            pl.BlockSpec((1, D), lambda i: (0, 0)),
        ],
        out_specs=pl.BlockSpec((_TCB, D), lambda i: (i, 0)),
        out_shape=jax.ShapeDtypeStruct((N, D), jnp.float32),
    )(acc, cnt, x, wl, wr, b)


def kernel(x, edge_index, W1l, W1r, b1, W2l, W2r, b2):
    src3 = edge_index[0].reshape(NW, NCHUNK, CH)
    dst3 = edge_index[1].reshape(NW, NCHUNK, CH)
    dst3c = edge_index[1].reshape(NW, NCHUNKC, CHC)
    zacc = jnp.zeros((RPT, D), jnp.float32)
    zc = jnp.zeros((RPT, CW), jnp.float32)
    on1 = jnp.ones((CHC, CW), jnp.float32)

    cnt1 = _sc_cnt(dst3c, zc, on1)
    acc1 = _sc_agg(x, src3, dst3, zacc)
    h = _tc_layer(acc1, cnt1, x, W1l, W1r, b1.reshape(1, D))
    acc2 = _sc_agg(h, src3, dst3, zacc)
    return _tc_layer(acc2, cnt1, h, W2l, W2r, b2.reshape(1, D))
